# baseline (speedup 1.0000x reference)
import dataclasses

import jax
import jax.numpy as jnp
from jax import lax
from jax.experimental import pallas as pl
from jax.experimental.pallas import tpu as pltpu
from jax.experimental.pallas import tpu_sc as plsc

NC = 2
NS = 16
NW = NC * NS
L = 16
K = 128
D = 128
SPLIT0 = 0.35


def _compiler_params():
    cp = pltpu.CompilerParams()
    if "needs_layout_passes" in pltpu.CompilerParams.__dataclass_fields__:
        cp = dataclasses.replace(cp, needs_layout_passes=False)
    return cp


_MESH = plsc.VectorSubcoreMesh(core_axis_name="c", subcore_axis_name="s")


def _make_agg(nrows, npad, compose_rows):
    rows_per_tile = npad // NS
    compose = compose_rows > 0
    n0 = (int(nrows * SPLIT0) // (NS * 8)) * 8
    rows0 = n0 * NS
    n1 = (nrows - rows0) // NS
    assert n1 % 8 == 0 and rows0 + n1 * NS == nrows
    nmax = max(n0, n1)

    scratch = [
        pltpu.VMEM((nmax, K), jnp.int32),
        pltpu.VMEM((K,), jnp.int32),
        pltpu.VMEM((1, K), jnp.int32),
        pltpu.VMEM((K, D), jnp.float32),
        pltpu.VMEM_SHARED((npad, D), jnp.float32),
        pltpu.SemaphoreType.DMA,
    ]
    if compose:
        scratch.insert(0, pltpu.VMEM((compose_rows,), jnp.int32))

    def body(*refs):
        if compose:
            (table_hbm, edge_hbm, cncpt_hbm, out_hbm,
             cncpt_v, edge_v, ib, db, rv, acc, sem) = refs
        else:
            (table_hbm, edge_hbm, out_hbm,
             edge_v, ib, db, rv, acc, sem) = refs
        c = lax.axis_index("c")
        s = lax.axis_index("s")

        if compose:
            pltpu.sync_copy(cncpt_hbm, cncpt_v)

        @pl.loop(0, K)
        def _(i):
            @pl.loop(0, D, step=L)
            def _(jj):
                rv[i, pl.ds(jj, L)] = jnp.zeros((L,), jnp.float32)

        @pl.loop(0, rows_per_tile, step=K)
        def _(r):
            pltpu.sync_copy(rv, acc.at[pl.ds(s * rows_per_tile + r, K)])

        plsc.subcore_barrier()

        def run(nmine, start):
            pltpu.sync_copy(edge_hbm.at[pl.ds(start, nmine)],
                            edge_v.at[pl.ds(0, nmine)])

            @pl.loop(0, nmine)
            def _(j):
                @pl.loop(0, K, step=L)
                def _(i):
                    word = edge_v[j, pl.ds(i, L)]
                    sidx = lax.bitwise_and(word, 0xFFFF)
                    if compose:
                        sidx = plsc.load_gather(cncpt_v, [sidx])
                    ib[pl.ds(i, L)] = sidx
                    db[0, pl.ds(i, L)] = lax.shift_right_logical(word, 16)
                pltpu.async_copy(table_hbm.at[ib], rv, sem).wait()
                pltpu.sync_copy(rv, acc.at[db.at[0]], add=True)

        @pl.when(c == 0)
        def _():
            run(n0, s * n0)

        @pl.when(c == 1)
        def _():
            run(n1, rows0 + s * n1)

        plsc.subcore_barrier()
        pltpu.sync_copy(
            acc.at[pl.ds(s * rows_per_tile, rows_per_tile)],
            out_hbm.at[c, pl.ds(s * rows_per_tile, rows_per_tile)],
        )

    return pl.kernel(
        body,
        out_type=jax.ShapeDtypeStruct((NC, npad, D), jnp.float32),
        mesh=_MESH,
        scratch_types=scratch,
        compiler_params=_compiler_params(),
    )


def _tc_linear_relu(p, w, b, npad):
    br = 1024

    def body(p_ref, w_ref, b_ref, o_ref):
        x = p_ref[0] + p_ref[1]
        y = lax.dot_general(
            x, w_ref[...], (((1,), (1,)), ((), ())),
            preferred_element_type=jnp.float32,
        )
        o_ref[...] = jnp.maximum(y + b_ref[...], 0.0)

    return pl.pallas_call(
        body,
        grid=(npad // br,),
        in_specs=[
            pl.BlockSpec((NC, br, D), lambda i: (0, i, 0)),
            pl.BlockSpec((D, D), lambda i: (0, 0)),
            pl.BlockSpec((1, D), lambda i: (0, 0)),
        ],
        out_specs=pl.BlockSpec((br, D), lambda i: (i, 0)),
        out_shape=jax.ShapeDtypeStruct((npad, D), jnp.float32),
    )(p, w, b)


def kernel(cncpt_ids, edge_index, emb_table, W1, b1, W2, b2):
    n = cncpt_ids.shape[0]
    e = edge_index.shape[1]

    npad = -(-(n + 1) // (NS * K)) * (NS * K)
    nrows = -(-e // (NS * 8 * K)) * (NS * 8)
    epad = nrows * K

    src = edge_index[0].astype(jnp.int32)
    dst = edge_index[1].astype(jnp.int32)
    packed = jnp.bitwise_or(src, jnp.left_shift(dst, 16))
    packed = jnp.concatenate(
        [packed, jnp.full((epad - e,), n << 16, jnp.int32)])
    packed = packed.reshape(nrows, K)
    cids = cncpt_ids.astype(jnp.int32)

    p1 = _make_agg(nrows, npad, compose_rows=n)(emb_table, packed, cids)
    h1 = _tc_linear_relu(p1, W1, b1.reshape(1, D), npad)

    p2 = _make_agg(nrows, npad, compose_rows=0)(h1, packed)
    h2 = _tc_linear_relu(p2, W2, b2.reshape(1, D), npad)
    return h2[:n]


# speedup vs baseline: 1.0700x; 1.0700x over previous
import dataclasses

import jax
import jax.numpy as jnp
from jax import lax
from jax.experimental import pallas as pl
from jax.experimental.pallas import tpu as pltpu
from jax.experimental.pallas import tpu_sc as plsc

NC = 2
NS = 16
NW = NC * NS
L = 16
K = 128
D = 128
SPLIT0 = 0.5


def _compiler_params():
    cp = pltpu.CompilerParams()
    if "needs_layout_passes" in pltpu.CompilerParams.__dataclass_fields__:
        cp = dataclasses.replace(cp, needs_layout_passes=False)
    return cp


_MESH = plsc.VectorSubcoreMesh(core_axis_name="c", subcore_axis_name="s")


def _make_agg(nrows, npad, compose_rows):
    rows_per_tile = npad // NS
    compose = compose_rows > 0
    n0 = (int(nrows * SPLIT0) // (NS * 8)) * 8
    rows0 = n0 * NS
    n1 = (nrows - rows0) // NS
    assert n1 % 8 == 0 and rows0 + n1 * NS == nrows
    nmax = max(n0, n1)

    scratch = [
        pltpu.VMEM((nmax, K), jnp.int32),
        pltpu.VMEM((K,), jnp.int32),
        pltpu.VMEM((1, K), jnp.int32),
        pltpu.VMEM((K, D), jnp.float32),
        pltpu.VMEM_SHARED((npad, D), jnp.float32),
        pltpu.SemaphoreType.DMA,
    ]
    if compose:
        scratch.insert(0, pltpu.VMEM((compose_rows,), jnp.int32))

    def body(*refs):
        if compose:
            (table_hbm, edge_hbm, cncpt_hbm, out_hbm,
             cncpt_v, edge_v, ib, db, rv, acc, sem) = refs
        else:
            (table_hbm, edge_hbm, out_hbm,
             edge_v, ib, db, rv, acc, sem) = refs
        c = lax.axis_index("c")
        s = lax.axis_index("s")

        if compose:
            pltpu.sync_copy(cncpt_hbm, cncpt_v)

        @pl.loop(0, K)
        def _(i):
            @pl.loop(0, D, step=L)
            def _(jj):
                rv[i, pl.ds(jj, L)] = jnp.zeros((L,), jnp.float32)

        @pl.loop(0, rows_per_tile, step=K)
        def _(r):
            pltpu.sync_copy(rv, acc.at[pl.ds(s * rows_per_tile + r, K)])

        plsc.subcore_barrier()

        def run(nmine, start):
            pltpu.sync_copy(edge_hbm.at[pl.ds(start, nmine)],
                            edge_v.at[pl.ds(0, nmine)])

            @pl.loop(0, nmine)
            def _(j):
                @pl.loop(0, K, step=L)
                def _(i):
                    word = edge_v[j, pl.ds(i, L)]
                    sidx = lax.bitwise_and(word, 0xFFFF)
                    if compose:
                        sidx = plsc.load_gather(cncpt_v, [sidx])
                    ib[pl.ds(i, L)] = sidx
                    db[0, pl.ds(i, L)] = lax.shift_right_logical(word, 16)
                pltpu.async_copy(table_hbm.at[ib], rv, sem).wait()
                pltpu.sync_copy(rv, acc.at[db.at[0]], add=True)

        @pl.when(c == 0)
        def _():
            run(n0, s * n0)

        @pl.when(c == 1)
        def _():
            run(n1, rows0 + s * n1)

        plsc.subcore_barrier()
        pltpu.sync_copy(
            acc.at[pl.ds(s * rows_per_tile, rows_per_tile)],
            out_hbm.at[c, pl.ds(s * rows_per_tile, rows_per_tile)],
        )

    return pl.kernel(
        body,
        out_type=jax.ShapeDtypeStruct((NC, npad, D), jnp.float32),
        mesh=_MESH,
        scratch_types=scratch,
        compiler_params=_compiler_params(),
    )


def _tc_linear_relu(p, w, b, npad):
    br = 1024

    def body(p_ref, w_ref, b_ref, o_ref):
        x = p_ref[0] + p_ref[1]
        y = lax.dot_general(
            x, w_ref[...], (((1,), (1,)), ((), ())),
            preferred_element_type=jnp.float32,
        )
        o_ref[...] = jnp.maximum(y + b_ref[...], 0.0)

    return pl.pallas_call(
        body,
        grid=(npad // br,),
        in_specs=[
            pl.BlockSpec((NC, br, D), lambda i: (0, i, 0)),
            pl.BlockSpec((D, D), lambda i: (0, 0)),
            pl.BlockSpec((1, D), lambda i: (0, 0)),
        ],
        out_specs=pl.BlockSpec((br, D), lambda i: (i, 0)),
        out_shape=jax.ShapeDtypeStruct((npad, D), jnp.float32),
    )(p, w, b)


def kernel(cncpt_ids, edge_index, emb_table, W1, b1, W2, b2):
    n = cncpt_ids.shape[0]
    e = edge_index.shape[1]

    npad = -(-(n + 1) // (NS * K)) * (NS * K)
    nrows = -(-e // (NS * 8 * K)) * (NS * 8)
    epad = nrows * K

    src = edge_index[0].astype(jnp.int32)
    dst = edge_index[1].astype(jnp.int32)
    packed = jnp.bitwise_or(src, jnp.left_shift(dst, 16))
    packed = jnp.concatenate(
        [packed, jnp.full((epad - e,), n << 16, jnp.int32)])
    packed = packed.reshape(nrows, K)
    cids = cncpt_ids.astype(jnp.int32)

    p1 = _make_agg(nrows, npad, compose_rows=n)(emb_table, packed, cids)
    h1 = _tc_linear_relu(p1, W1, b1.reshape(1, D), npad)

    p2 = _make_agg(nrows, npad, compose_rows=0)(h1, packed)
    h2 = _tc_linear_relu(p2, W2, b2.reshape(1, D), npad)
    return h2[:n]


# speedup vs baseline: 1.5460x; 1.4448x over previous
import dataclasses

import jax
import jax.numpy as jnp
from jax import lax
from jax.experimental import pallas as pl
from jax.experimental.pallas import tpu as pltpu
from jax.experimental.pallas import tpu_sc as plsc

NC = 2
NS = 16
NW = NC * NS
L = 16
K = 128
D = 128


def _compiler_params():
    cp = pltpu.CompilerParams()
    if "needs_layout_passes" in pltpu.CompilerParams.__dataclass_fields__:
        cp = dataclasses.replace(cp, needs_layout_passes=False)
    return cp


_MESH = plsc.VectorSubcoreMesh(core_axis_name="c", subcore_axis_name="s")


def _make_agg(nchunks, npad, compose_rows):
    rows_per_tile = npad // NS
    compose = compose_rows > 0

    scratch = [
        pltpu.VMEM((nchunks, K), jnp.int32),
        pltpu.VMEM((nchunks, K), jnp.int32),
        pltpu.VMEM((K,), jnp.int32),
        pltpu.VMEM((K, D), jnp.float32),
        pltpu.VMEM_SHARED((npad, D), jnp.float32),
        pltpu.SemaphoreType.DMA,
    ]
    if compose:
        scratch.insert(0, pltpu.VMEM((compose_rows,), jnp.int32))

    def body(*refs):
        if compose:
            (table_hbm, src_hbm, dst_hbm, cncpt_hbm, out_hbm,
             cncpt_v, src_v, dst_v, idx_v, rows_v, acc, sem) = refs
        else:
            (table_hbm, src_hbm, dst_hbm, out_hbm,
             src_v, dst_v, idx_v, rows_v, acc, sem) = refs
        c = lax.axis_index("c")
        s = lax.axis_index("s")
        w = s * NC + c

        pltpu.sync_copy(src_hbm.at[w], src_v)
        pltpu.sync_copy(dst_hbm.at[w], dst_v)
        if compose:
            pltpu.sync_copy(cncpt_hbm, cncpt_v)

        @pl.loop(0, K)
        def _(i):
            @pl.loop(0, D, step=L)
            def _(j):
                rows_v[i, pl.ds(j, L)] = jnp.zeros((L,), jnp.float32)

        @pl.loop(0, rows_per_tile, step=K)
        def _(r):
            pltpu.sync_copy(rows_v, acc.at[pl.ds(s * rows_per_tile + r, K)])

        plsc.subcore_barrier()

        @pl.loop(0, nchunks)
        def _(j):
            if compose:
                @pl.loop(0, K, step=L)
                def _(i):
                    sidx = src_v[j, pl.ds(i, L)]
                    idx_v[pl.ds(i, L)] = plsc.load_gather(cncpt_v, [sidx])
                gidx = idx_v
            else:
                gidx = src_v.at[j]
            pltpu.async_copy(table_hbm.at[gidx], rows_v, sem).wait()
            pltpu.sync_copy(rows_v, acc.at[dst_v.at[j]], add=True)

        plsc.subcore_barrier()
        pltpu.sync_copy(
            acc.at[pl.ds(s * rows_per_tile, rows_per_tile)],
            out_hbm.at[c, pl.ds(s * rows_per_tile, rows_per_tile)],
        )

    return pl.kernel(
        body,
        out_type=jax.ShapeDtypeStruct((NC, npad, D), jnp.float32),
        mesh=_MESH,
        scratch_types=scratch,
        compiler_params=_compiler_params(),
    )


def _tc_linear_relu(p, w, b, npad):
    br = 1024

    def body(p_ref, w_ref, b_ref, o_ref):
        x = p_ref[0] + p_ref[1]
        y = lax.dot_general(
            x, w_ref[...], (((1,), (1,)), ((), ())),
            preferred_element_type=jnp.float32,
        )
        o_ref[...] = jnp.maximum(y + b_ref[...], 0.0)

    return pl.pallas_call(
        body,
        grid=(npad // br,),
        in_specs=[
            pl.BlockSpec((NC, br, D), lambda i: (0, i, 0)),
            pl.BlockSpec((D, D), lambda i: (0, 0)),
            pl.BlockSpec((1, D), lambda i: (0, 0)),
        ],
        out_specs=pl.BlockSpec((br, D), lambda i: (i, 0)),
        out_shape=jax.ShapeDtypeStruct((npad, D), jnp.float32),
    )(p, w, b)


def kernel(cncpt_ids, edge_index, emb_table, W1, b1, W2, b2):
    n = cncpt_ids.shape[0]
    e = edge_index.shape[1]

    npad = -(-(n + 1) // (NS * K)) * (NS * K)
    nchunks = -(-e // (NW * K))
    epad = NW * nchunks * K

    src = edge_index[0].astype(jnp.int32)
    dst = edge_index[1].astype(jnp.int32)
    srcp = jnp.concatenate([src, jnp.zeros((epad - e,), jnp.int32)])
    dstp = jnp.concatenate([dst, jnp.full((epad - e,), n, jnp.int32)])
    srcp = srcp.reshape(NW, nchunks, K)
    dstp = dstp.reshape(NW, nchunks, K)
    cids = cncpt_ids.astype(jnp.int32)

    p1 = _make_agg(nchunks, npad, compose_rows=n)(emb_table, srcp, dstp, cids)
    h1 = _tc_linear_relu(p1, W1, b1.reshape(1, D), npad)

    p2 = _make_agg(nchunks, npad, compose_rows=0)(h1, srcp, dstp)
    h2 = _tc_linear_relu(p2, W2, b2.reshape(1, D), npad)
    return h2[:n]
